# baseline (speedup 1.0000x reference)
import functools

import jax
import jax.numpy as jnp
from jax import lax
from jax.experimental import pallas as pl
from jax.experimental.pallas import tpu as pltpu
from jax.experimental.pallas import tpu_sc as plsc

BATCH = 16384
HIST = 200
EMBED_DIM = 32
VOCAB = 1000000

_B = BATCH * HIST
_NC, _NS = 2, 16
_NW = _NC * _NS
_IW = 128
_UNITS = _B // _IW
_U_PER_W = _UNITS // _NW
_K = 8
_CHUNK = _IW * _K
_N_CHUNKS = _U_PER_W // _K
_HT = HIST // 8
_BT = BATCH // 128
_CT = EMBED_DIM // 8
_NTR = 4


def _emb_kernel(idx_hbm, tab_hbm, out_hbm, idx_v, rows_v, tr_v, idx_sems,
                g_sems, wb_sems):
    wid = lax.axis_index("s") * _NC + lax.axis_index("c")
    r0 = wid * _U_PER_W

    def idx_copy(c):
        buf = lax.rem(c, 3)
        return pltpu.make_async_copy(
            idx_hbm.at[pl.ds(r0 + c * _K, _K)],
            idx_v.at[buf],
            idx_sems.at[buf],
        )

    def gather(c, j):
        buf = lax.rem(c, 2)
        ibuf = lax.rem(c, 3)
        return pltpu.make_async_copy(
            tab_hbm.at[idx_v.at[ibuf, j]],
            rows_v.at[buf, pl.ds(j * _IW, _IW)],
            g_sems.at[buf],
        )

    def writeback(g, t, ct):
        ht = lax.div(g, 1024)
        bt = lax.rem(lax.div(g, 8), 128)
        hs = lax.rem(g, 8)
        return pltpu.make_async_copy(
            tr_v.at[t, pl.ds(ct * 8, 8)],
            out_hbm.at[ht * 8 + hs, ct, bt],
            wb_sems.at[t],
        )

    def fire_gathers(c):
        idx_copy(c).wait()
        for j in range(_K):
            gather(c, j).start()

    iota = lax.iota(jnp.int32, 16)

    idx_copy(0).start()
    idx_copy(1).start()
    fire_gathers(0)

    @pl.loop(0, _N_CHUNKS)
    def _chunk(c):
        buf = lax.rem(c, 2)

        @pl.when(c + 2 < _N_CHUNKS)
        def _():
            idx_copy(c + 2).start()

        @pl.when(c + 1 < _N_CHUNKS)
        def _():
            fire_gathers(c + 1)

        for j in range(_K):
            gather(c, j).wait()

        @pl.loop(0, _K)
        def _unit(u):
            ul = c * _K + u
            g = r0 + ul
            t = lax.rem(ul, _NTR)

            @pl.when(ul >= _NTR)
            def _():
                for ct in range(_CT):
                    writeback(g - _NTR, t, ct).wait()

            base = u * _IW
            for cc in range(EMBED_DIM):
                col = jnp.full((16,), cc, jnp.int32)
                for l in range(8):
                    rowi = iota + (base + l * 16)
                    v = plsc.load_gather(rows_v.at[buf], [rowi, col])
                    tr_v[t, cc, pl.ds(l * 16, 16)] = v

            for ct in range(_CT):
                writeback(g, t, ct).start()

    last = r0 + _U_PER_W
    for d in range(_NTR):
        for ct in range(_CT):
            writeback(last - _NTR + d, d, ct).wait()


def kernel(inputs, table):
    idx = (
        inputs.T.reshape(_HT, 8, _BT, 128)
        .transpose(0, 2, 1, 3)
        .reshape(_UNITS, _IW)
    )
    mesh = plsc.VectorSubcoreMesh(core_axis_name="c", subcore_axis_name="s")
    run = functools.partial(
        pl.kernel,
        out_type=jax.ShapeDtypeStruct((HIST, _CT, _BT, 8, 128), jnp.float32),
        mesh=mesh,
        scratch_types=[
            pltpu.VMEM((3, _K, _IW), jnp.int32),
            pltpu.VMEM((2, _CHUNK, EMBED_DIM), jnp.float32),
            pltpu.VMEM((_NTR, EMBED_DIM, 128), jnp.float32),
            pltpu.SemaphoreType.DMA((3,)),
            pltpu.SemaphoreType.DMA((2,)),
            pltpu.SemaphoreType.DMA((_NTR,)),
        ],
        compiler_params=pltpu.CompilerParams(
            use_tc_tiling_on_sc=False, needs_layout_passes=False
        ),
    )(_emb_kernel)
    out = run(idx, table)
    return (
        out.transpose(2, 4, 0, 1, 3).reshape(BATCH, HIST, EMBED_DIM)
    )


# speedup vs baseline: 1.1874x; 1.1874x over previous
import functools

import jax
import jax.numpy as jnp
from jax import lax
from jax.experimental import pallas as pl
from jax.experimental.pallas import tpu as pltpu
from jax.experimental.pallas import tpu_sc as plsc

BATCH = 16384
HIST = 200
EMBED_DIM = 32
VOCAB = 1000000

_B = BATCH * HIST
_NC, _NS = 2, 16
_NW = _NC * _NS
_IW = 128
_UNITS = _B // _IW
_U_PER_W = _UNITS // _NW
_K = 8
_CHUNK = _IW * _K
_N_CHUNKS = _U_PER_W // _K
_B_PER_W = _U_PER_W * _IW
_HT = HIST // 8
_BT = BATCH // 128
_CT = EMBED_DIM // 8
_NBUF = 3


def _emb_kernel(idx_hbm, tab_hbm, out_hbm, idx_v, rows_v, idx_sems, g_sems,
                wb_sems):
    wid = lax.axis_index("s") * _NC + lax.axis_index("c")
    irow_base = wid * _U_PER_W
    row_base = wid * _B_PER_W

    def idx_copy(c):
        buf = lax.rem(c, _NBUF)
        return pltpu.make_async_copy(
            idx_hbm.at[pl.ds(irow_base + c * _K, _K)],
            idx_v.at[buf],
            idx_sems.at[buf],
        )

    def gather(c, j):
        buf = lax.rem(c, _NBUF)
        return pltpu.make_async_copy(
            tab_hbm.at[idx_v.at[buf, j]],
            rows_v.at[buf, pl.ds(j * _IW, _IW)],
            g_sems.at[buf],
        )

    def writeback(c):
        buf = lax.rem(c, _NBUF)
        return pltpu.make_async_copy(
            rows_v.at[buf],
            out_hbm.at[pl.ds(row_base + c * _CHUNK, _CHUNK)],
            wb_sems.at[buf],
        )

    def fire_gathers(c):
        idx_copy(c).wait()
        for j in range(_K):
            gather(c, j).start()

    idx_copy(0).start()
    idx_copy(1).start()
    fire_gathers(0)

    @pl.loop(0, _N_CHUNKS)
    def _chunk(c):
        @pl.when(c + 2 < _N_CHUNKS)
        def _():
            idx_copy(c + 2).start()

        @pl.when(c + 1 < _N_CHUNKS)
        def _():
            @pl.when(c >= 2)
            def _():
                writeback(c - 2).wait()

            fire_gathers(c + 1)

        for j in range(_K):
            gather(c, j).wait()
        writeback(c).start()

    writeback(_N_CHUNKS - 3).wait()
    writeback(_N_CHUNKS - 2).wait()
    writeback(_N_CHUNKS - 1).wait()


def _tr_kernel(y_ref, z_ref):
    t = y_ref[...].transpose(0, 2, 1)
    t5 = t.reshape(16, 8, 4, 8, 128)
    for hs in range(8):
        for ct in range(_CT):
            z_ref[hs, ct] = t5[:, hs, ct]


def kernel(inputs, table):
    idx = (
        inputs.T.reshape(_HT, 8, _BT, 128)
        .transpose(0, 2, 1, 3)
        .reshape(_UNITS, _IW)
    )
    mesh = plsc.VectorSubcoreMesh(core_axis_name="c", subcore_axis_name="s")
    run = functools.partial(
        pl.kernel,
        out_type=jax.ShapeDtypeStruct((_B, EMBED_DIM), jnp.float32),
        mesh=mesh,
        scratch_types=[
            pltpu.VMEM((_NBUF, _K, _IW), jnp.int32),
            pltpu.VMEM((_NBUF, _CHUNK, EMBED_DIM), jnp.float32),
            pltpu.SemaphoreType.DMA((_NBUF,)),
            pltpu.SemaphoreType.DMA((_NBUF,)),
            pltpu.SemaphoreType.DMA((_NBUF,)),
        ],
        compiler_params=pltpu.CompilerParams(use_tc_tiling_on_sc=False),
    )(_emb_kernel)
    y = run(idx, table)

    z = pl.pallas_call(
        _tr_kernel,
        out_shape=jax.ShapeDtypeStruct((HIST, _CT, _BT, 8, 128), jnp.float32),
        grid=(_HT, _BT // 16),
        in_specs=[
            pl.BlockSpec((128, _IW, EMBED_DIM), lambda ht, g: (ht * 8 + g, 0, 0)),
        ],
        out_specs=pl.BlockSpec(
            (8, _CT, 16, 8, 128), lambda ht, g: (ht, 0, g, 0, 0)
        ),
    )(y.reshape(_UNITS, _IW, EMBED_DIM))

    return z.transpose(2, 4, 0, 1, 3).reshape(BATCH, HIST, EMBED_DIM)
